# baseline (speedup 1.0000x reference)
import functools

import jax
import jax.numpy as jnp
from jax import lax
from jax.experimental import pallas as pl
from jax.experimental.pallas import tpu as pltpu
from jax.experimental.pallas import tpu_sc as plsc

N = 10000
E = 160000
D_IN = 256
D_HID = 512
CH = 128
NCH = D_HID // CH
NC = 2
NS = 16
K = 125
KP = 80
EPT = E // NS
NB_P = EPT // KP
NB_D = E // (NC * NS) // K
RPT = N // NS
ALPHA = 0.1

_mesh = plsc.VectorSubcoreMesh(core_axis_name="c", subcore_axis_name="s")


CH_D = 128


@functools.partial(
    pl.kernel,
    mesh=_mesh,
    out_type=jax.ShapeDtypeStruct((NC, NS, RPT, CH_D), jnp.float32),
    scratch_types=[
        pltpu.VMEM((NB_D, K), jnp.int32),
        pltpu.VMEM((K, CH_D), jnp.float32),
        pltpu.VMEM((K, CH_D), jnp.float32),
        pltpu.VMEM_SHARED((N, CH_D), jnp.float32),
        pltpu.SemaphoreType.DMA,
    ],
)
def _deg_kernel(col_hbm, deg_out, col_v, ones_v, zer_v, acc, sem):
    cid = lax.axis_index("c")
    sid = lax.axis_index("s")
    wid = cid * NS + sid
    pltpu.sync_copy(col_hbm.at[wid], col_v)

    def _fill(i, carry):
        for j in range(CH_D // 16):
            ones_v[i, pl.ds(j * 16, 16)] = jnp.ones((16,), jnp.float32)
            zer_v[i, pl.ds(j * 16, 16)] = jnp.zeros((16,), jnp.float32)
        return carry

    lax.fori_loop(0, K, _fill, 0)

    def _zero(z, carry):
        pltpu.sync_copy(zer_v, acc.at[pl.ds(sid * RPT + z * K, K)])
        return carry

    lax.fori_loop(0, RPT // K, _zero, 0)
    plsc.subcore_barrier()

    def _scatter(b, carry):
        pltpu.sync_copy(ones_v, acc.at[col_v.at[b]], add=True)
        return carry

    lax.fori_loop(0, NB_D, _scatter, 0)
    plsc.subcore_barrier()
    pltpu.sync_copy(acc.at[pl.ds(sid * RPT, RPT)], deg_out.at[cid, sid])


@functools.partial(
    pl.kernel,
    mesh=_mesh,
    out_type=[jax.ShapeDtypeStruct((NS, RPT, CH), jnp.float32)] * NCH,
    scratch_types=[
        pltpu.VMEM((EPT,), jnp.int32),
        pltpu.VMEM((NB_P, KP), jnp.int32),
        pltpu.VMEM((KP, CH), jnp.float32),
        pltpu.VMEM((KP, CH), jnp.float32),
        pltpu.VMEM_SHARED((N, CH), jnp.float32),
        pltpu.SemaphoreType.DMA,
        pltpu.SemaphoreType.DMA,
        pltpu.SemaphoreType.DMA,
        pltpu.SemaphoreType.DMA,
    ],
)
def _prop_kernel(row_hbm, col_hbm, hs0, hs1, hs2, hs3,
                 a0, a1, a2, a3, row_v, col_v, buf0, buf1, acc,
                 sem0, sem1, ssem0, ssem1):
    cid = lax.axis_index("c")
    sid = lax.axis_index("s")
    pltpu.sync_copy(row_hbm.at[pl.ds(sid * EPT, EPT)], row_v)
    pltpu.sync_copy(col_hbm.at[sid], col_v)

    def _ridx(b):
        return row_v.at[pl.ds(b * KP, KP)]

    def _zero_own_rows():
        def _fill_zer(i, carry):
            for j in range(CH // 16):
                buf0[i, pl.ds(j * 16, 16)] = jnp.zeros((16,), jnp.float32)
            return carry

        lax.fori_loop(0, KP, _fill_zer, 0)

        def _zero(z, carry):
            pltpu.sync_copy(buf0, acc.at[pl.ds(sid * RPT + z * KP, KP)])
            return carry

        lax.fori_loop(0, RPT // KP, _zero, 0)
        rem = RPT - (RPT // KP) * KP
        if rem:
            pltpu.sync_copy(buf0.at[pl.ds(0, rem)],
                            acc.at[pl.ds(sid * RPT + RPT - rem, rem)])

    hs_list = [hs0, hs1, hs2, hs3]
    out_list = [a0, a1, a2, a3]
    CPC = NCH // NC
    for c in range(NCH):
        @pl.when(cid == c // CPC)
        def _(c=c):
            hs = hs_list[c]
            if c % CPC == 0:
                _zero_own_rows()
            plsc.subcore_barrier()

            pltpu.async_copy(hs.at[_ridx(0)], buf0, sem0)

            def _edge_pair(g, carry):
                b0 = 2 * g
                pltpu.make_async_copy(hs.at[_ridx(b0)], buf0, sem0).wait()
                pltpu.async_copy(hs.at[_ridx(b0 + 1)], buf1, sem1)
                pltpu.sync_copy(buf0, acc.at[col_v.at[b0]], add=True)
                pltpu.async_copy(hs.at[_ridx(b0 + 2)], buf0, sem0)
                pltpu.make_async_copy(hs.at[_ridx(b0 + 1)], buf1, sem1).wait()
                pltpu.sync_copy(buf1, acc.at[col_v.at[b0 + 1]], add=True)
                return carry

            lax.fori_loop(0, NB_P // 2, _edge_pair, 0)
            pltpu.make_async_copy(hs.at[_ridx(NB_P - 1)], buf0, sem0).wait()
            pltpu.sync_copy(buf0, acc.at[col_v.at[NB_P - 1]], add=True)
            plsc.subcore_barrier()
            pltpu.sync_copy(acc.at[pl.ds(sid * RPT, RPT)], out_list[c].at[sid])
            if c % CPC != CPC - 1:
                _zero_own_rows()


_BM = 1000
_GM = N // _BM


def _tc_mm_body(x_ref, w_ref, b_ref, h_ref):
    h_ref[...] = lax.dot_general(x_ref[...], w_ref[...], (((1,), (1,)), ((), ())),
                                 preferred_element_type=jnp.float32) + b_ref[...]


def _tc_mm(x, w, b2):
    return pl.pallas_call(
        _tc_mm_body,
        grid=(_GM,),
        in_specs=[
            pl.BlockSpec((_BM, D_IN), lambda m: (m, 0)),
            pl.BlockSpec((D_HID, D_IN), lambda m: (0, 0)),
            pl.BlockSpec((1, D_HID), lambda m: (0, 0)),
        ],
        out_specs=pl.BlockSpec((_BM, D_HID), lambda m: (m, 0)),
        out_shape=jax.ShapeDtypeStruct((N, D_HID), jnp.float32),
    )(x, w, b2)


def _tc_scale_body(deg_ref, h_ref, dinv_ref, *hs_refs):
    deg = jnp.sum(deg_ref[...], axis=0)[:, 0:1] + 1.0
    dinv = lax.rsqrt(deg)
    dinv_ref[...] = dinv
    hs = h_ref[...] * dinv
    for c in range(NCH):
        hs_refs[c][...] = hs[:, c * CH:(c + 1) * CH]


def _tc_scale(deg, h):
    return pl.pallas_call(
        _tc_scale_body,
        grid=(_GM,),
        in_specs=[
            pl.BlockSpec((NC, _BM, CH_D), lambda m: (0, m, 0)),
            pl.BlockSpec((_BM, D_HID), lambda m: (m, 0)),
        ],
        out_specs=[
            pl.BlockSpec((_BM, 1), lambda m: (m, 0)),
        ] + [pl.BlockSpec((_BM, CH), lambda m: (m, 0))] * NCH,
        out_shape=[
            jax.ShapeDtypeStruct((N, 1), jnp.float32),
        ] + [jax.ShapeDtypeStruct((N, CH), jnp.float32)] * NCH,
    )(deg, h)


def _mix_body(dinv_ref, h_ref, a0, a1, a2, a3, out_ref, *hs_refs):
    dinv = dinv_ref[...]
    agg = jnp.concatenate([a0[...], a1[...], a2[...], a3[...]], axis=1)
    out = (1.0 - ALPHA) * dinv * agg + ((1.0 - ALPHA) * dinv * dinv + ALPHA) * h_ref[...]
    out_ref[...] = out
    if hs_refs:
        hs = dinv * out
        for c in range(NCH):
            hs_refs[c][...] = hs[:, c * CH:(c + 1) * CH]


def _mix(dinv, h, aggs, want_hs):
    n_hs = NCH if want_hs else 0
    return pl.pallas_call(
        _mix_body,
        grid=(_GM,),
        in_specs=[
            pl.BlockSpec((_BM, 1), lambda m: (m, 0)),
            pl.BlockSpec((_BM, D_HID), lambda m: (m, 0)),
        ] + [pl.BlockSpec((_BM, CH), lambda m: (m, 0))] * NCH,
        out_specs=[pl.BlockSpec((_BM, D_HID), lambda m: (m, 0))]
        + [pl.BlockSpec((_BM, CH), lambda m: (m, 0))] * n_hs,
        out_shape=[jax.ShapeDtypeStruct((N, D_HID), jnp.float32)]
        + [jax.ShapeDtypeStruct((N, CH), jnp.float32)] * n_hs,
    )(dinv, h, *aggs)


def kernel(x, edge_index, W, b):
    ei = edge_index.astype(jnp.int32)
    row = ei[0]
    col = ei[1].reshape(NS, NB_P, KP)
    col_d = ei[1].reshape(NC * NS, NB_D, K)

    deg_p = _deg_kernel(col_d)
    h = _tc_mm(x, W, b.reshape(1, D_HID))
    dinv, *hs = _tc_scale(deg_p.reshape(NC, N, CH_D), h)
    agg1 = _prop_kernel(row, col, *hs)
    x1, *hs2 = _mix(dinv, h, [a.reshape(N, CH) for a in agg1], want_hs=True)
    agg2 = _prop_kernel(row, col, *hs2)
    (x2,) = _mix(dinv, x1, [a.reshape(N, CH) for a in agg2], want_hs=False)
    return (x1, x2)
